# baseline (speedup 1.0000x reference)
import functools

import jax
import jax.numpy as jnp
from jax.experimental import pallas as pl
from jax.experimental.pallas import tpu as pltpu

BS, T, DIM, ND, NC = 1024, 16, 512, 365, 24
NDP, NCP, NCLS = 384, 32, 128
BLK = 256
NB = BS // BLK


def _dot(a, b):
    return jax.lax.dot_general(a, b, (((1,), (0,)), ((), ())),
                               preferred_element_type=jnp.float32)


def _dot_t(a, b):
    return jax.lax.dot_general(a, b, (((0,), (0,)), ((), ())),
                               preferred_element_type=jnp.float32)


def _prelu(x, a):
    return jnp.where(x >= 0, x, a * x)


def _body(frame_ref, vd_ref, dv_ref, sub_ref, ev_ref, dc_ref, gt_ref,
          Wc2d_ref, bc2d_ref, ac2d_ref,
          Wv2d_ref, bv2d_ref, av2d_ref,
          Wd2v_ref, bd2v_ref, ad2v_ref,
          Wd2v2_ref, bd2v2_ref, ad2v2_ref,
          Wsa_ref, bsa_ref, qsa_ref,
          Wfc_ref, bfc_ref,
          Wcls_ref, bcls_ref,
          loss_ref, idx_ref,
          video_s, d2v_s, acc_s, sw_s, c2d_s, aw2_s,
          vdp_s, dvp_s, clsp_s, bclsp_s, dcp_s, ewp_s, lsum_s):
    p = pl.program_id(0)
    i = pl.program_id(1)

    @pl.when(jnp.logical_and(p == 0, i == 0))
    def _init():
        vdp_s[:] = jnp.zeros_like(vdp_s)
        dvp_s[:] = jnp.zeros_like(dvp_s)
        vdp_s[:, 0:ND] = vd_ref[:]
        dvp_s[:, 0:ND] = dv_ref[:]
        clsp_s[:] = jnp.zeros_like(clsp_s)
        clsp_s[:, 0:NC] = Wcls_ref[:]
        bclsp_s[:] = jnp.full_like(bclsp_s, -1e30)
        bclsp_s[:, 0:NC] = bcls_ref[:]

        acc_s[:] = jnp.zeros_like(acc_s)
        sw_s[:] = jnp.zeros_like(sw_s)
        sw_s[0:ND, :] = _dot(sub_ref[:], Wd2v_ref[:])
        ewp_s[:] = jnp.zeros_like(ewp_s)
        ewp_s[0:NC, :] = _dot(ev_ref[:], Wc2d_ref[:])
        dcp_s[:] = jnp.zeros_like(dcp_s)
        dcp_s[0:ND, 0:NC] = dc_ref[:]
        c2d_s[:] = _prelu(_dot(dcp_s[:], ewp_s[:]) + bc2d_ref[:],
                          ac2d_ref[0, 0])

    @pl.when(p == 0)
    def _phase0():
        v = jnp.mean(frame_ref[:], axis=1)
        video_s[pl.ds(i * BLK, BLK), :] = v
        vW = _dot(v, Wv2d_ref[:])
        acc_s[:] += _dot_t(dvp_s[pl.ds(i * BLK, BLK), :], vW)
        d2v_s[pl.ds(i * BLK, BLK), :] = _prelu(
            _dot(vdp_s[pl.ds(i * BLK, BLK), :], sw_s[:]) + bd2v_ref[:],
            ad2v_ref[0, 0])

    @pl.when(jnp.logical_and(p == 1, i == 0))
    def _transition():
        c2d = c2d_s[:]
        v2d = _prelu(acc_s[:] + bv2d_ref[:], av2d_ref[0, 0])
        qsa = qsa_ref[:]
        mask = jax.lax.broadcasted_iota(jnp.int32, (NDP, DIM // 4), 0) < ND
        hc = jnp.tanh(_dot(c2d, Wsa_ref[:]) + bsa_ref[:])
        hv = jnp.tanh(_dot(v2d, Wsa_ref[:]) + bsa_ref[:])
        sc = jnp.sum(jnp.where(mask, hc * qsa, 0.0)) / ND
        sv = jnp.sum(jnp.where(mask, hv * qsa, 0.0)) / ND
        m = jnp.maximum(sc, sv)
        e0, e1 = jnp.exp(sc - m), jnp.exp(sv - m)
        att = (e0 * c2d + e1 * v2d) / (e0 + e1)
        aw2_s[:] = _dot(att, Wd2v2_ref[:])
        lsum_s[0, 0] = 0.0

    @pl.when(p == 1)
    def _phase1():
        d2v2 = _prelu(_dot(vdp_s[pl.ds(i * BLK, BLK), :], aw2_s[:])
                      + bd2v2_ref[:], ad2v2_ref[0, 0])
        vc = (_dot(d2v2, Wfc_ref[0:DIM, :])
              + _dot(d2v_s[pl.ds(i * BLK, BLK), :], Wfc_ref[DIM:2 * DIM, :])
              + _dot(video_s[pl.ds(i * BLK, BLK), :],
                     Wfc_ref[2 * DIM:3 * DIM, :])
              + bfc_ref[:])
        preds = _dot(vc, clsp_s[:]) + bclsp_s[:]
        mx = jnp.max(preds, axis=1, keepdims=True)
        z = preds - mx
        lse = jnp.log(jnp.sum(jnp.exp(z), axis=1, keepdims=True))
        cls_ids = jax.lax.broadcasted_iota(jnp.int32, (BLK, NCLS), 1)
        z_gt = jnp.sum(jnp.where(cls_ids == gt_ref[:], z, 0.0), axis=1,
                       keepdims=True)
        lsum_s[0, 0] += jnp.sum(z_gt - lse)
        idx_ref[:] = jnp.min(jnp.where(preds == mx, cls_ids, NCLS), axis=1,
                             keepdims=True)

    @pl.when(jnp.logical_and(p == 1, i == NB - 1))
    def _final():
        loss_ref[:] = jnp.full((1, 1), -1.0 / BS) * lsum_s[0, 0]


@functools.partial(jax.jit, static_argnames=())
def kernel(frame_emb, cd_adj, dc_adj, vd_adj, dv_adj, subevent, event,
           logit_scale, ground_truth, W_c2d, b_c2d, a_c2d, W_v2d, b_v2d,
           a_v2d, W_d2v, b_d2v, a_d2v, W_d2v2, b_d2v2, a_d2v2, W_sa, b_sa,
           q_sa, W_fc, b_fc, W_cls, b_cls):
    del cd_adj, logit_scale

    f32 = jnp.float32
    gt2 = ground_truth.reshape(BS, 1)
    s = lambda x: x.reshape(1, 1)
    r = lambda x: x.reshape(1, -1)

    def full_spec(shape):
        nd = len(shape)
        return pl.BlockSpec(shape, lambda p, i, _n=nd: (0,) * _n)

    loss2, idx = pl.pallas_call(
        _body,
        grid=(2, NB),
        in_specs=[
            pl.BlockSpec((BLK, T, DIM),
                         lambda p, i: (i * (1 - p) + (NB - 1) * p, 0, 0)),
            full_spec((BS, ND)),
            full_spec((BS, ND)),
            full_spec((ND, DIM)),
            full_spec((NC, DIM)),
            full_spec((ND, NC)),
            pl.BlockSpec((BLK, 1), lambda p, i: (i, 0)),
            full_spec((DIM, DIM)), full_spec((1, DIM)), full_spec((1, 1)),
            full_spec((DIM, DIM)), full_spec((1, DIM)), full_spec((1, 1)),
            full_spec((DIM, DIM)), full_spec((1, DIM)), full_spec((1, 1)),
            full_spec((DIM, DIM)), full_spec((1, DIM)), full_spec((1, 1)),
            full_spec((DIM, DIM // 4)), full_spec((1, DIM // 4)),
            full_spec((1, DIM // 4)),
            full_spec((3 * DIM, DIM)), full_spec((1, DIM)),
            full_spec((DIM, NC)), full_spec((1, NC)),
        ],
        out_specs=(
            pl.BlockSpec((1, 1), lambda p, i: (0, 0)),
            pl.BlockSpec((BLK, 1), lambda p, i: (i, 0)),
        ),
        out_shape=(jax.ShapeDtypeStruct((1, 1), f32),
                   jax.ShapeDtypeStruct((BS, 1), jnp.int32)),
        scratch_shapes=[
            pltpu.VMEM((BS, DIM), f32),
            pltpu.VMEM((BS, DIM), f32),
            pltpu.VMEM((NDP, DIM), f32),
            pltpu.VMEM((NDP, DIM), f32),
            pltpu.VMEM((NDP, DIM), f32),
            pltpu.VMEM((NDP, DIM), f32),
            pltpu.VMEM((BS, NDP), f32),
            pltpu.VMEM((BS, NDP), f32),
            pltpu.VMEM((DIM, NCLS), f32),
            pltpu.VMEM((1, NCLS), f32),
            pltpu.VMEM((NDP, NCP), f32),
            pltpu.VMEM((NCP, DIM), f32),
            pltpu.SMEM((1, 1), f32),
        ],
    )(frame_emb, vd_adj, dv_adj, subevent, event, dc_adj, gt2,
      W_c2d, r(b_c2d), s(a_c2d),
      W_v2d, r(b_v2d), s(a_v2d),
      W_d2v, r(b_d2v), s(a_d2v),
      W_d2v2, r(b_d2v2), s(a_d2v2),
      W_sa, b_sa, q_sa,
      W_fc, r(b_fc),
      W_cls, r(b_cls))

    return loss2[0, 0], idx


# speedup vs baseline: 1.0148x; 1.0148x over previous
import functools

import jax
import jax.numpy as jnp
from jax.experimental import pallas as pl
from jax.experimental.pallas import tpu as pltpu

BS, T, DIM, ND, NC = 1024, 16, 512, 365, 24
NDP, NCP, NCLS = 384, 32, 128
BLK = 256
NB = BS // BLK


def _dot(a, b):
    return jax.lax.dot_general(a, b, (((1,), (0,)), ((), ())),
                               preferred_element_type=jnp.float32)


def _dot_t(a, b):
    return jax.lax.dot_general(a, b, (((0,), (0,)), ((), ())),
                               preferred_element_type=jnp.float32)


def _prelu(x, a):
    return jnp.where(x >= 0, x, a * x)


def _body(frame_ref, vd_ref, dv_ref, sub_ref, ev_ref, dc_ref, gt_ref,
          Wc2d_ref, bc2d_ref, ac2d_ref,
          Wv2d_ref, bv2d_ref, av2d_ref,
          Wd2v_ref, bd2v_ref, ad2v_ref,
          Wd2v2_ref, bd2v2_ref, ad2v2_ref,
          Wsa_ref, bsa_ref, qsa_ref,
          Wfc_ref, bfc_ref,
          Wcls_ref, bcls_ref,
          loss_ref, idx_ref,
          vcp_s, acc_s, sw_s, c2d_s, aw2_s,
          vdp_s, dvp_s, clsp_s, bclsp_s, dcp_s, ewp_s, lsum_s):
    p = pl.program_id(0)
    i = pl.program_id(1)

    @pl.when(jnp.logical_and(p == 0, i == 0))
    def _init():
        vdp_s[:] = jnp.zeros_like(vdp_s)
        dvp_s[:] = jnp.zeros_like(dvp_s)
        vdp_s[:, 0:ND] = vd_ref[:]
        dvp_s[:, 0:ND] = dv_ref[:]
        clsp_s[:] = jnp.zeros_like(clsp_s)
        clsp_s[:, 0:NC] = Wcls_ref[:]
        bclsp_s[:] = jnp.full_like(bclsp_s, -1e30)
        bclsp_s[:, 0:NC] = bcls_ref[:]

        acc_s[:] = jnp.zeros_like(acc_s)
        sw_s[:] = jnp.zeros_like(sw_s)
        sw_s[0:ND, :] = _dot(sub_ref[:], Wd2v_ref[:])
        ewp_s[:] = jnp.zeros_like(ewp_s)
        ewp_s[0:NC, :] = _dot(ev_ref[:], Wc2d_ref[:])
        dcp_s[:] = jnp.zeros_like(dcp_s)
        dcp_s[0:ND, 0:NC] = dc_ref[:]
        c2d_s[:] = _prelu(_dot(dcp_s[:], ewp_s[:]) + bc2d_ref[:],
                          ac2d_ref[0, 0])

    @pl.when(p == 0)
    def _phase0():
        v = jnp.mean(frame_ref[:], axis=1)
        vW = _dot(v, Wv2d_ref[:])
        acc_s[:] += _dot_t(dvp_s[pl.ds(i * BLK, BLK), :], vW)
        d2v = _prelu(
            _dot(vdp_s[pl.ds(i * BLK, BLK), :], sw_s[:]) + bd2v_ref[:],
            ad2v_ref[0, 0])
        vcp_s[pl.ds(i * BLK, BLK), :] = (
            _dot(d2v, Wfc_ref[DIM:2 * DIM, :])
            + _dot(v, Wfc_ref[2 * DIM:3 * DIM, :]))

    @pl.when(jnp.logical_and(p == 1, i == 0))
    def _transition():
        c2d = c2d_s[:]
        v2d = _prelu(acc_s[:] + bv2d_ref[:], av2d_ref[0, 0])
        qsa = qsa_ref[:]
        mask = jax.lax.broadcasted_iota(jnp.int32, (NDP, DIM // 4), 0) < ND
        hc = jnp.tanh(_dot(c2d, Wsa_ref[:]) + bsa_ref[:])
        hv = jnp.tanh(_dot(v2d, Wsa_ref[:]) + bsa_ref[:])
        sc = jnp.sum(jnp.where(mask, hc * qsa, 0.0)) / ND
        sv = jnp.sum(jnp.where(mask, hv * qsa, 0.0)) / ND
        m = jnp.maximum(sc, sv)
        e0, e1 = jnp.exp(sc - m), jnp.exp(sv - m)
        att = (e0 * c2d + e1 * v2d) / (e0 + e1)
        aw2_s[:] = _dot(att, Wd2v2_ref[:])
        lsum_s[0, 0] = 0.0

    @pl.when(p == 1)
    def _phase1():
        d2v2 = _prelu(_dot(vdp_s[pl.ds(i * BLK, BLK), :], aw2_s[:])
                      + bd2v2_ref[:], ad2v2_ref[0, 0])
        vc = (_dot(d2v2, Wfc_ref[0:DIM, :])
              + vcp_s[pl.ds(i * BLK, BLK), :] + bfc_ref[:])
        preds = _dot(vc, clsp_s[:]) + bclsp_s[:]
        mx = jnp.max(preds, axis=1, keepdims=True)
        z = preds - mx
        lse = jnp.log(jnp.sum(jnp.exp(z), axis=1, keepdims=True))
        cls_ids = jax.lax.broadcasted_iota(jnp.int32, (BLK, NCLS), 1)
        z_gt = jnp.sum(jnp.where(cls_ids == gt_ref[:], z, 0.0), axis=1,
                       keepdims=True)
        lsum_s[0, 0] += jnp.sum(z_gt - lse)
        idx_ref[:] = jnp.min(jnp.where(preds == mx, cls_ids, NCLS), axis=1,
                             keepdims=True)

    @pl.when(jnp.logical_and(p == 1, i == NB - 1))
    def _final():
        loss_ref[:] = jnp.full((1, 1), -1.0 / BS) * lsum_s[0, 0]


@functools.partial(jax.jit, static_argnames=())
def kernel(frame_emb, cd_adj, dc_adj, vd_adj, dv_adj, subevent, event,
           logit_scale, ground_truth, W_c2d, b_c2d, a_c2d, W_v2d, b_v2d,
           a_v2d, W_d2v, b_d2v, a_d2v, W_d2v2, b_d2v2, a_d2v2, W_sa, b_sa,
           q_sa, W_fc, b_fc, W_cls, b_cls):
    del cd_adj, logit_scale

    f32 = jnp.float32
    gt2 = ground_truth.reshape(BS, 1)
    s = lambda x: x.reshape(1, 1)
    r = lambda x: x.reshape(1, -1)

    def full_spec(shape):
        nd = len(shape)
        return pl.BlockSpec(shape, lambda p, i, _n=nd: (0,) * _n)

    loss2, idx = pl.pallas_call(
        _body,
        grid=(2, NB),
        in_specs=[
            pl.BlockSpec((BLK, T, DIM),
                         lambda p, i: (i * (1 - p) + (NB - 1) * p, 0, 0)),
            full_spec((BS, ND)),
            full_spec((BS, ND)),
            full_spec((ND, DIM)),
            full_spec((NC, DIM)),
            full_spec((ND, NC)),
            pl.BlockSpec((BLK, 1), lambda p, i: (i, 0)),
            full_spec((DIM, DIM)), full_spec((1, DIM)), full_spec((1, 1)),
            full_spec((DIM, DIM)), full_spec((1, DIM)), full_spec((1, 1)),
            full_spec((DIM, DIM)), full_spec((1, DIM)), full_spec((1, 1)),
            full_spec((DIM, DIM)), full_spec((1, DIM)), full_spec((1, 1)),
            full_spec((DIM, DIM // 4)), full_spec((1, DIM // 4)),
            full_spec((1, DIM // 4)),
            full_spec((3 * DIM, DIM)), full_spec((1, DIM)),
            full_spec((DIM, NC)), full_spec((1, NC)),
        ],
        out_specs=(
            pl.BlockSpec((1, 1), lambda p, i: (0, 0)),
            pl.BlockSpec((BLK, 1), lambda p, i: (i, 0)),
        ),
        out_shape=(jax.ShapeDtypeStruct((1, 1), f32),
                   jax.ShapeDtypeStruct((BS, 1), jnp.int32)),
        scratch_shapes=[
            pltpu.VMEM((BS, DIM), f32),
            pltpu.VMEM((NDP, DIM), f32),
            pltpu.VMEM((NDP, DIM), f32),
            pltpu.VMEM((NDP, DIM), f32),
            pltpu.VMEM((NDP, DIM), f32),
            pltpu.VMEM((BS, NDP), f32),
            pltpu.VMEM((BS, NDP), f32),
            pltpu.VMEM((DIM, NCLS), f32),
            pltpu.VMEM((1, NCLS), f32),
            pltpu.VMEM((NDP, NCP), f32),
            pltpu.VMEM((NCP, DIM), f32),
            pltpu.SMEM((1, 1), f32),
        ],
    )(frame_emb, vd_adj, dv_adj, subevent, event, dc_adj, gt2,
      W_c2d, r(b_c2d), s(a_c2d),
      W_v2d, r(b_v2d), s(a_v2d),
      W_d2v, r(b_d2v), s(a_d2v),
      W_d2v2, r(b_d2v2), s(a_d2v2),
      W_sa, b_sa, q_sa,
      W_fc, r(b_fc),
      W_cls, r(b_cls))

    return loss2[0, 0], idx
